# baseline (speedup 1.0000x reference)
import jax
import jax.numpy as jnp
from jax.experimental import pallas as pl


def _vq_block(x_ref, w_ref, loss_ref, q_ref, enc_ref, idx_ref):
    i = pl.program_id(0)
    xb = x_ref[0].T
    w = w_ref[...]
    x_norm = jnp.sum(xb * xb, axis=1, keepdims=True)
    w_norm = jnp.sum(w * w, axis=1)
    xw = jax.lax.dot_general(xb, w, (((1,), (1,)), ((), ())),
                             preferred_element_type=jnp.float32)
    dist = (x_norm + w_norm[None, :]) - 2.0 * xw
    k_iota = jax.lax.broadcasted_iota(jnp.int32, dist.shape, 1)
    dmin = jnp.min(dist, axis=1, keepdims=True)
    idx = jnp.min(jnp.where(dist == dmin, k_iota, dist.shape[1]),
                  axis=1).astype(jnp.int32)
    onehot = (k_iota == idx[:, None]).astype(jnp.float32)
    q = jax.lax.dot_general(onehot, w, (((1,), (0,)), ((), ())),
                            preferred_element_type=jnp.float32)
    diff = q - xb
    lp = jnp.sum(diff * diff)

    @pl.when(i == 0)
    def _init():
        loss_ref[...] = jnp.zeros_like(loss_ref)

    loss_ref[...] = loss_ref[...] + lp
    q_ref[0] = q.T
    enc_ref[0] = onehot
    idx_ref[0, 0] = idx


def kernel(x, W):
    B, C, H, Wd = x.shape
    K, D = W.shape
    R = H * Wd
    x3 = x.reshape(B, D, R)
    loss_acc, q3, enc3, idx3 = pl.pallas_call(
        _vq_block,
        grid=(B,),
        in_specs=[
            pl.BlockSpec((1, D, R), lambda i: (i, 0, 0)),
            pl.BlockSpec((K, D), lambda i: (0, 0)),
        ],
        out_specs=[
            pl.BlockSpec((1, 1), lambda i: (0, 0)),
            pl.BlockSpec((1, D, R), lambda i: (i, 0, 0)),
            pl.BlockSpec((1, R, K), lambda i: (i, 0, 0)),
            pl.BlockSpec((1, 1, R), lambda i: (i, 0, 0)),
        ],
        out_shape=[
            jax.ShapeDtypeStruct((1, 1), jnp.float32),
            jax.ShapeDtypeStruct((B, D, R), jnp.float32),
            jax.ShapeDtypeStruct((B, R, K), jnp.float32),
            jax.ShapeDtypeStruct((B, 1, R), jnp.int32),
        ],
    )(x3, W)
    loss = 2.0 * loss_acc[0, 0] / (B * R * D)
    out = q3.reshape(B, D, H, Wd)
    encodings = enc3.reshape(B * R, K)
    encoding_indices = idx3.reshape(B * R)
    return (loss, out, encodings, encoding_indices)


# speedup vs baseline: 1.0127x; 1.0127x over previous
import jax
import jax.numpy as jnp
from jax.experimental import pallas as pl


def _vq_block(x_ref, w_ref, loss_ref, q_ref, enc_ref, idx_ref):
    i = pl.program_id(0)
    xbT = x_ref[0]
    w = w_ref[...]
    x_norm = jnp.sum(xbT * xbT, axis=0, keepdims=True).T
    w_norm = jnp.sum(w * w, axis=1)
    xw = jax.lax.dot_general(xbT, w, (((0,), (1,)), ((), ())),
                             preferred_element_type=jnp.float32)
    dist = (x_norm + w_norm[None, :]) - 2.0 * xw
    k_iota = jax.lax.broadcasted_iota(jnp.int32, dist.shape, 1)
    dmin = jnp.min(dist, axis=1, keepdims=True)
    idx = jnp.min(jnp.where(dist == dmin, k_iota, dist.shape[1]),
                  axis=1).astype(jnp.int32)
    onehot = (k_iota == idx[:, None]).astype(jnp.float32)
    qT = jax.lax.dot_general(w, onehot, (((0,), (1,)), ((), ())),
                             preferred_element_type=jnp.float32)
    diff = qT - xbT
    lp = jnp.sum(diff * diff)

    @pl.when(i == 0)
    def _init():
        loss_ref[...] = jnp.zeros_like(loss_ref)

    loss_ref[...] = loss_ref[...] + lp
    q_ref[0] = qT
    enc_ref[0] = onehot
    idx_ref[0, 0] = idx


def kernel(x, W):
    B, C, H, Wd = x.shape
    K, D = W.shape
    R = H * Wd
    x3 = x.reshape(B, D, R)
    loss_acc, q3, enc3, idx3 = pl.pallas_call(
        _vq_block,
        grid=(B,),
        in_specs=[
            pl.BlockSpec((1, D, R), lambda i: (i, 0, 0)),
            pl.BlockSpec((K, D), lambda i: (0, 0)),
        ],
        out_specs=[
            pl.BlockSpec((1, 1), lambda i: (0, 0)),
            pl.BlockSpec((1, D, R), lambda i: (i, 0, 0)),
            pl.BlockSpec((1, R, K), lambda i: (i, 0, 0)),
            pl.BlockSpec((1, 1, R), lambda i: (i, 0, 0)),
        ],
        out_shape=[
            jax.ShapeDtypeStruct((1, 1), jnp.float32),
            jax.ShapeDtypeStruct((B, D, R), jnp.float32),
            jax.ShapeDtypeStruct((B, R, K), jnp.float32),
            jax.ShapeDtypeStruct((B, 1, R), jnp.int32),
        ],
    )(x3, W)
    loss = 2.0 * loss_acc[0, 0] / (B * R * D)
    out = q3.reshape(B, D, H, Wd)
    encodings = enc3.reshape(B * R, K)
    encoding_indices = idx3.reshape(B * R)
    return (loss, out, encodings, encoding_indices)


# speedup vs baseline: 1.4873x; 1.4687x over previous
import jax
import jax.numpy as jnp
from jax.experimental import pallas as pl


def _vq_block(x_ref, w_ref, loss_ref, q_ref, enc_ref, idx_ref):
    i = pl.program_id(0)
    xb = x_ref[0]
    w = w_ref[...]
    x_norm = jnp.sum(xb * xb, axis=1, keepdims=True)
    w_norm = jnp.sum(w * w, axis=1)
    xw2 = jax.lax.dot_general(xb, w + w, (((1,), (1,)), ((), ())),
                              preferred_element_type=jnp.float32)
    dist = (x_norm + w_norm[None, :]) - xw2
    k_iota = jax.lax.broadcasted_iota(jnp.int32, dist.shape, 1)
    dmin = jnp.min(dist, axis=1, keepdims=True)
    sel = jnp.where(dist == dmin, k_iota, dist.shape[1])
    idx = jnp.min(sel, axis=1).astype(jnp.int32)
    onehot = (sel == idx[:, None]).astype(jnp.float32)
    q = jax.lax.dot_general(onehot, w, (((1,), (0,)), ((), ())),
                            preferred_element_type=jnp.float32)
    diff = q - xb
    lp = jnp.sum(diff * diff)

    @pl.when(i == 0)
    def _init():
        loss_ref[...] = jnp.zeros_like(loss_ref)

    loss_ref[...] = loss_ref[...] + lp
    q_ref[0] = q
    enc_ref[0] = onehot
    idx_ref[0, 0] = idx


def kernel(x, W):
    B, C, H, Wd = x.shape
    K, D = W.shape
    R = H * Wd
    xp = jnp.transpose(x, (0, 2, 3, 1))
    x3 = xp.reshape(B, R, D)
    loss_acc, q3, enc3, idx3 = pl.pallas_call(
        _vq_block,
        grid=(B,),
        in_specs=[
            pl.BlockSpec((1, R, D), lambda i: (i, 0, 0)),
            pl.BlockSpec((K, D), lambda i: (0, 0)),
        ],
        out_specs=[
            pl.BlockSpec((1, 1), lambda i: (0, 0)),
            pl.BlockSpec((1, R, D), lambda i: (i, 0, 0)),
            pl.BlockSpec((1, R, K), lambda i: (i, 0, 0)),
            pl.BlockSpec((1, 1, R), lambda i: (i, 0, 0)),
        ],
        out_shape=[
            jax.ShapeDtypeStruct((1, 1), jnp.float32),
            jax.ShapeDtypeStruct((B, R, D), jnp.float32),
            jax.ShapeDtypeStruct((B, R, K), jnp.float32),
            jax.ShapeDtypeStruct((B, 1, R), jnp.int32),
        ],
    )(x3, W)
    loss = 2.0 * loss_acc[0, 0] / (B * R * D)
    out = q3.reshape(B, H, Wd, D).transpose(0, 3, 1, 2)
    encodings = enc3.reshape(B * R, K)
    encoding_indices = idx3.reshape(B * R)
    return (loss, out, encodings, encoding_indices)


# speedup vs baseline: 1.5937x; 1.0715x over previous
import jax
import jax.numpy as jnp
from jax.experimental import pallas as pl


def _vq_block(x_ref, w_ref, loss_ref, q_ref, enc_ref, idx_ref):
    i = pl.program_id(0)
    xb = x_ref[0]
    w = w_ref[...]
    x_norm = jnp.sum(xb * xb, axis=1, keepdims=True)
    w_norm = jnp.sum(w * w, axis=1)
    xw2 = jax.lax.dot_general(xb, w + w, (((1,), (1,)), ((), ())),
                              preferred_element_type=jnp.float32)
    dist = (x_norm + w_norm[None, :]) - xw2
    k_iota = jax.lax.broadcasted_iota(jnp.int32, dist.shape, 1
                                      ).astype(jnp.float32)
    dmin = jnp.min(dist, axis=1, keepdims=True)
    sel = jnp.where(dist == dmin, k_iota, float(dist.shape[1]))
    idxf = jnp.min(sel, axis=1, keepdims=True)
    onehot = (sel == idxf).astype(jnp.float32)
    idx = idxf[:, 0].astype(jnp.int32)
    q = jax.lax.dot_general(onehot, w, (((1,), (0,)), ((), ())),
                            preferred_element_type=jnp.float32)
    diff = q - xb
    lp = jnp.sum(diff * diff)

    @pl.when(i == 0)
    def _init():
        loss_ref[...] = jnp.zeros_like(loss_ref)

    loss_ref[...] = loss_ref[...] + lp
    q_ref[0] = q
    enc_ref[0] = onehot
    idx_ref[0, 0] = idx


def kernel(x, W):
    B, C, H, Wd = x.shape
    K, D = W.shape
    R = H * Wd
    xp = jnp.transpose(x, (0, 2, 3, 1))
    x3 = xp.reshape(B, R, D)
    loss_acc, q3, enc3, idx3 = pl.pallas_call(
        _vq_block,
        grid=(B,),
        in_specs=[
            pl.BlockSpec((1, R, D), lambda i: (i, 0, 0)),
            pl.BlockSpec((K, D), lambda i: (0, 0)),
        ],
        out_specs=[
            pl.BlockSpec((1, 1), lambda i: (0, 0)),
            pl.BlockSpec((1, R, D), lambda i: (i, 0, 0)),
            pl.BlockSpec((1, R, K), lambda i: (i, 0, 0)),
            pl.BlockSpec((1, 1, R), lambda i: (i, 0, 0)),
        ],
        out_shape=[
            jax.ShapeDtypeStruct((1, 1), jnp.float32),
            jax.ShapeDtypeStruct((B, R, D), jnp.float32),
            jax.ShapeDtypeStruct((B, R, K), jnp.float32),
            jax.ShapeDtypeStruct((B, 1, R), jnp.int32),
        ],
    )(x3, W)
    loss = 2.0 * loss_acc[0, 0] / (B * R * D)
    out = q3.reshape(B, H, Wd, D).transpose(0, 3, 1, 2)
    encodings = enc3.reshape(B * R, K)
    encoding_indices = idx3.reshape(B * R)
    return (loss, out, encodings, encoding_indices)
